# Initial kernel scaffold; baseline (speedup 1.0000x reference)
import jax
import jax.numpy as jnp
from jax.experimental import pallas as pl


def kernel(x, table):
    raise NotImplementedError("write your pallas kernel here")


# speedup vs baseline: 3.3674x; 3.3674x over previous
import functools

import jax
import jax.numpy as jnp
import numpy as np
from jax import lax
from jax.experimental import pallas as pl
from jax.experimental.pallas import tpu as pltpu
from jax.experimental.pallas import tpu_sc as plsc

_VOCAB = 100000
_D = 64
_SEQ = 200
_BATCH = 4096

_NC = 2
_NS = 16
_NW = _NC * _NS
_ROWS = _BATCH * _SEQ
_RPW = _ROWS // _NW
_C = 2 * _SEQ
_NCHUNK = _RPW // _C
_G = 80
_NG = _C // _G
_VECS = _D // 16


def _positional_encoding() -> jnp.ndarray:
    pos = np.arange(_SEQ, dtype=np.float64)[:, None]
    div = np.exp(np.arange(0, _D, 2, dtype=np.float64) * (-np.log(10000.0) / _D))
    pe = np.zeros((_SEQ, _D), dtype=np.float32)
    pe[:, 0::2] = np.sin(pos * div).astype(np.float32)
    pe[:, 1::2] = np.cos(pos * div).astype(np.float32)
    return jnp.asarray(np.tile(pe, (_C // _SEQ, 1)))


_MESH = plsc.VectorSubcoreMesh(core_axis_name="c", subcore_axis_name="s")


@functools.partial(
    pl.kernel,
    mesh=_MESH,
    out_type=jax.ShapeDtypeStruct((_ROWS, _D), jnp.float32),
    scratch_types=[
        pltpu.VMEM((_C,), jnp.int32),
        pltpu.VMEM((_C, _D), jnp.float32),
        pltpu.VMEM((_C, _D), jnp.float32),
        pltpu.SemaphoreType.DMA,
    ],
    compiler_params=pltpu.CompilerParams(use_tc_tiling_on_sc=False),
)
def _embed(idx_hbm, table_hbm, pe_hbm, out_hbm, idx_v, rows_v, pe_v, sem):
    wid = lax.axis_index("s") * _NC + lax.axis_index("c")
    base = wid * _RPW
    pltpu.sync_copy(pe_hbm, pe_v)

    def chunk(c, carry):
        off = base + c * _C
        pltpu.sync_copy(idx_hbm.at[pl.ds(off, _C)], idx_v)
        copies = []
        for g in range(_NG):
            copies.append(
                pltpu.async_copy(
                    table_hbm.at[idx_v.at[pl.ds(g * _G, _G)]],
                    rows_v.at[pl.ds(g * _G, _G)],
                    sem,
                )
            )
        for cp in copies:
            cp.wait()

        def row(i, rcarry):
            for j in range(_VECS):
                sl = pl.ds(j * 16, 16)
                rows_v[i, sl] = rows_v[i, sl] + pe_v[i, sl]
            return rcarry

        lax.fori_loop(0, _C, row, 0)
        pltpu.sync_copy(rows_v, out_hbm.at[pl.ds(off, _C)])
        return carry

    lax.fori_loop(0, _NCHUNK, chunk, 0)


def kernel(x, table):
    idx = x.reshape(-1).astype(jnp.int32)
    out = _embed(idx, table, _positional_encoding())
    return out.reshape(_BATCH, _SEQ, _D)


# speedup vs baseline: 3.5860x; 1.0649x over previous
import functools

import jax
import jax.numpy as jnp
import numpy as np
from jax import lax
from jax.experimental import pallas as pl
from jax.experimental.pallas import tpu as pltpu
from jax.experimental.pallas import tpu_sc as plsc

_VOCAB = 100000
_D = 64
_SEQ = 200
_BATCH = 4096

_NC = 2
_NS = 16
_NW = _NC * _NS
_ROWS = _BATCH * _SEQ
_RPW = _ROWS // _NW
_C = 2 * _SEQ
_NCHUNK = _RPW // _C
_G = 80
_NG = _C // _G
_VECS = _D // 16


def _positional_encoding() -> jnp.ndarray:
    pos = np.arange(_SEQ, dtype=np.float64)[:, None]
    div = np.exp(np.arange(0, _D, 2, dtype=np.float64) * (-np.log(10000.0) / _D))
    pe = np.zeros((_SEQ, _D), dtype=np.float32)
    pe[:, 0::2] = np.sin(pos * div).astype(np.float32)
    pe[:, 1::2] = np.cos(pos * div).astype(np.float32)
    return jnp.asarray(np.tile(pe, (_C // _SEQ, 1)))


_MESH = plsc.VectorSubcoreMesh(core_axis_name="c", subcore_axis_name="s")


@functools.partial(
    pl.kernel,
    mesh=_MESH,
    out_type=jax.ShapeDtypeStruct((_ROWS, _D), jnp.float32),
    scratch_types=[
        pltpu.VMEM((_C,), jnp.int32),
        pltpu.VMEM((_C,), jnp.int32),
        pltpu.VMEM((_C, _D), jnp.float32),
        pltpu.VMEM((_C, _D), jnp.float32),
        pltpu.VMEM((_C, _D), jnp.float32),
        pltpu.SemaphoreType.DMA,
        pltpu.SemaphoreType.DMA,
        pltpu.SemaphoreType.DMA,
        pltpu.SemaphoreType.DMA,
    ],
    compiler_params=pltpu.CompilerParams(use_tc_tiling_on_sc=False),
)
def _embed(idx_hbm, table_hbm, pe_hbm, out_hbm,
           idx0, idx1, rows0, rows1, pe_v, sg0, sg1, so0, so1):
    wid = lax.axis_index("s") * _NC + lax.axis_index("c")
    base = wid * _RPW
    pltpu.sync_copy(pe_hbm, pe_v)
    slots = ((idx0, rows0, sg0, so0), (idx1, rows1, sg1, so1))

    def fire_gathers(idx_b, rows_b, sem):
        for g in range(_NG):
            pltpu.async_copy(
                table_hbm.at[idx_b.at[pl.ds(g * _G, _G)]],
                rows_b.at[pl.ds(g * _G, _G)],
                sem,
            )

    def wait_gathers(idx_b, rows_b, sem):
        for g in range(_NG):
            pltpu.make_async_copy(
                table_hbm.at[idx_b.at[pl.ds(g * _G, _G)]],
                rows_b.at[pl.ds(g * _G, _G)],
                sem,
            ).wait()

    def wait_out(rows_b, sem):
        pltpu.make_async_copy(rows_b, out_hbm.at[pl.ds(base, _C)], sem).wait()

    pltpu.sync_copy(idx_hbm.at[pl.ds(base, _C)], idx0)
    pltpu.sync_copy(idx_hbm.at[pl.ds(base + _C, _C)], idx1)
    fire_gathers(idx0, rows0, sg0)

    def body(ci, carry):
        for b in range(2):
            idx_b, rows_b, sg_b, so_b = slots[b]
            idx_n, rows_n, sg_n, so_n = slots[1 - b]
            cc = ci * 2 + b
            off = base + cc * _C
            wait_gathers(idx_b, rows_b, sg_b)

            def quad(i, _):
                r = i * 4
                for dr in range(4):
                    for j in range(_VECS):
                        sl = pl.ds(j * 16, 16)
                        rows_b[r + dr, sl] = rows_b[r + dr, sl] + pe_v[r + dr, sl]
                return 0

            lax.fori_loop(0, _C // 4, quad, 0)
            pltpu.async_copy(rows_b, out_hbm.at[pl.ds(off, _C)], so_b)

            @pl.when(cc + 2 < _NCHUNK)
            def _():
                pltpu.sync_copy(idx_hbm.at[pl.ds(off + 2 * _C, _C)], idx_b)

            @pl.when(cc + 1 < _NCHUNK)
            def _():
                @pl.when(cc >= 1)
                def _():
                    wait_out(rows_n, so_n)

                fire_gathers(idx_n, rows_n, sg_n)

        return carry

    lax.fori_loop(0, _NCHUNK // 2, body, 0)
    wait_out(rows0, so0)
    wait_out(rows1, so1)


def kernel(x, table):
    idx = x.reshape(-1).astype(jnp.int32)
    out = _embed(idx, table, _positional_encoding())
    return out.reshape(_BATCH, _SEQ, _D)
